# baseline (speedup 1.0000x reference)
import jax
import jax.numpy as jnp
from jax.experimental import pallas as pl
from jax.experimental.pallas import tpu as pltpu

_N = 10000
_IN_F = 128
_HID = 64
_OUT = 64
_ALPHA = 0.2
_NEGBIG = -1e30
_LOG2E = 1.4426950408889634

_BI = 1024
_BJ = 4096
_NI = (_N + _BI - 1) // _BI
_NJ = (_N + _BJ - 1) // _BJ
_BG = _BI // 8
_NP = _NI * _BG
_NC = _NJ * _BJ
_EXT = 128

_f32 = jnp.float32
_bf16 = jnp.bfloat16


def _row_t(a2, wh):
    return jax.lax.dot_general(a2, wh, (((0,), (1,)), ((), ())),
                               preferred_element_type=_f32)


def _extend(wh, rowv):
    b = wh.shape[0]
    ext = jnp.concatenate(
        [wh, jnp.ones((b, 1), _f32), jnp.zeros((b, _EXT - _HID - 1), _f32)],
        axis=1)
    return jnp.where(rowv, ext, 0.0).astype(_bf16)


def _prep_st(wh, a1, a2, colv):
    b = wh.shape[0]
    s = jnp.dot(wh, a1, preferred_element_type=_f32) * _LOG2E
    t = _row_t(a2, wh) * _LOG2E
    ones = jnp.ones((1, b), _f32)
    tp = jnp.concatenate([ones, jnp.where(colv, t, _NEGBIG)], axis=0).astype(_bf16)
    tb = jnp.concatenate([ones, jnp.where(colv, t * _ALPHA, _NEGBIG)], axis=0).astype(_bf16)
    tm = jnp.max(jnp.where(colv, t, _NEGBIG), keepdims=True)
    return s, tp, tb, tm


def _lrelu(z):
    return jnp.maximum(z, _ALPHA * z)


def _prelude_kernel(x_ref, wlin_ref, blin_ref, w0_ref, w1_ref,
                    a01_ref, a02_ref, a11_ref, a12_ref,
                    wh0_o, wh1_o, s0_o, s1_o, t0p_o, t0b_o, t1p_o, t1b_o,
                    t0m_o, t1m_o, t0m_s, t1m_s):
    i = pl.program_id(0)
    h = jnp.dot(x_ref[...], wlin_ref[...],
                preferred_element_type=_f32) + blin_ref[...]
    wh0 = jnp.dot(h, w0_ref[...], preferred_element_type=_f32)
    wh1 = jnp.dot(h, w1_ref[...], preferred_element_type=_f32)
    rowv = (i * _BI + jax.lax.broadcasted_iota(jnp.int32, (_BI, 1), 0)) < _N
    colv = (i * _BI + jax.lax.broadcasted_iota(jnp.int32, (1, _BI), 1)) < _N
    wh0_o[...] = _extend(wh0, rowv)
    wh1_o[...] = _extend(wh1, rowv)
    s0, t0p, t0b, bm0 = _prep_st(wh0, a01_ref[...], a02_ref[...], colv)
    s1, t1p, t1b, bm1 = _prep_st(wh1, a11_ref[...], a12_ref[...], colv)
    s0_o[...] = s0
    s1_o[...] = s1
    t0p_o[...] = t0p
    t0b_o[...] = t0b
    t1p_o[...] = t1p
    t1b_o[...] = t1b
    prev0 = jnp.where(i == 0, jnp.full((1, 1), _NEGBIG), t0m_s[...])
    prev1 = jnp.where(i == 0, jnp.full((1, 1), _NEGBIG), t1m_s[...])
    t0m_s[...] = jnp.maximum(prev0, bm0)
    t1m_s[...] = jnp.maximum(prev1, bm1)

    @pl.when(i == _NI - 1)
    def _():
        t0m_o[...] = t0m_s[...]
        t1m_o[...] = t1m_s[...]


def _head_step(okb, s_ref, tp_ref, tb_ref, tm_ref, wh_ref, acc):
    s = s_ref[...]
    m = _lrelu(s + tm_ref[...])
    b = s.shape[0]
    ones = jnp.ones((b, 1), _f32)
    ea = (s - m).astype(_bf16) + tp_ref[1:2, :]
    eb = (s * _ALPHA - m).astype(_bf16) + tb_ref[1:2, :]
    x = jnp.exp2(jnp.maximum(ea, eb))
    p = jnp.where(okb, x, _bf16(0.0))
    acc[...] += jnp.dot(p, wh_ref[...], preferred_element_type=_f32)


def _pass1_kernel(adj_ref, apack_ref, wh0_ref, wh1_ref, s0_ref, s1_ref,
                  t0p_ref, t0b_ref, t1p_ref, t1b_ref, t0m_ref, t1m_ref,
                  wend_ref, ae1_ref, ae2_ref,
                  whe_o, se_o, tep_o, teb_o, tem_o, pk_o,
                  acc0, acc1, tem_s):
    i = pl.program_id(0)
    j = pl.program_id(1)

    @pl.when(j == 0)
    def _():
        acc0[...] = jnp.zeros_like(acc0)
        acc1[...] = jnp.zeros_like(acc1)

    adjb = adj_ref[...] > 0

    pk = adj_ref[0:_BG, :]
    for r in range(1, 8):
        pk += adj_ref[r * _BG:(r + 1) * _BG, :] << r
    pk_o[...] = pk.astype(jnp.uint8)

    okb = adjb

    _head_step(okb, s0_ref, t0p_ref, t0b_ref, t0m_ref, wh0_ref, acc0)
    _head_step(okb, s1_ref, t1p_ref, t1b_ref, t1m_ref, wh1_ref, acc1)

    @pl.when(j == _NJ - 1)
    def _():
        h0 = acc0[:, :_HID] / jnp.maximum(acc0[:, _HID:_HID + 1], 1e-30)
        h1 = acc1[:, :_HID] / jnp.maximum(acc1[:, _HID:_HID + 1], 1e-30)
        x0 = jnp.where(h0 > 0, h0, jnp.exp(h0) - 1.0)
        x1 = jnp.where(h1 > 0, h1, jnp.exp(h1) - 1.0)
        whe = (jnp.dot(x0, wend_ref[:_HID, :], preferred_element_type=_f32)
               + jnp.dot(x1, wend_ref[_HID:, :], preferred_element_type=_f32))
        rowv = (i * _BI + jax.lax.broadcasted_iota(jnp.int32, (_BI, 1), 0)) < _N
        colv = (i * _BI + jax.lax.broadcasted_iota(jnp.int32, (1, _BI), 1)) < _N
        whe_o[...] = _extend(whe, rowv)
        se, tep, teb, bm = _prep_st(whe, ae1_ref[...], ae2_ref[...], colv)
        se_o[...] = se
        tep_o[...] = tep
        teb_o[...] = teb
        prev = jnp.where(i == 0, jnp.full((1, 1), _NEGBIG), tem_s[...])
        tem_s[...] = jnp.maximum(prev, bm)

        @pl.when(i == _NI - 1)
        def _():
            tem_o[...] = tem_s[...]


def _pass2_kernel(pk_ref, whe_ref, se_ref, tep_ref, teb_ref, tem_ref,
                  out_o, acc):
    j = pl.program_id(1)

    @pl.when(j == 0)
    def _():
        acc[...] = jnp.zeros_like(acc)

    pk = pk_ref[...].astype(jnp.int32)
    s = se_ref[...]
    m = _lrelu(s + tem_ref[...])
    sa = (s - m).astype(_bf16)
    sb = (s * _ALPHA - m).astype(_bf16)
    tp = tep_ref[1:2, :]
    tb = teb_ref[1:2, :]
    for r in range(8):
        lo, hi = r * _BG, (r + 1) * _BG
        ea = sa[lo:hi] + tp
        eb = sb[lo:hi] + tb
        x = jnp.exp2(jnp.maximum(ea, eb))
        p = jnp.where(((pk >> r) & 1) > 0, x, _bf16(0.0))
        acc[lo:hi, :] += jnp.dot(p, whe_ref[...],
                                 preferred_element_type=_f32)

    @pl.when(j == _NJ - 1)
    def _():
        o = acc[:, :_OUT] / jnp.maximum(acc[:, _OUT:_OUT + 1], 1e-30)
        z = o - jnp.max(o, axis=1, keepdims=True)
        pz = jnp.exp(z)
        out_o[...] = pz / jnp.sum(pz, axis=1, keepdims=True)


def kernel(x, adj, W_lin, b_lin, W_heads, a_heads, W_end, a_end):
    w0, w1 = W_heads[0], W_heads[1]
    a01, a02 = a_heads[0, :_HID], a_heads[0, _HID:]
    a11, a12 = a_heads[1, :_HID], a_heads[1, _HID:]
    ae1, ae2 = a_end[:_OUT], a_end[_OUT:]
    blin = b_lin.reshape(1, _IN_F)
    cols = jnp.arange(_BJ)
    apack = (((cols % _BG)[None, :] == jnp.arange(_BG)[:, None])
             * (2.0 ** (cols // _BG))[None, :]).astype(_bf16)

    const = lambda shape: pl.BlockSpec(shape, lambda *_: tuple(0 for _ in shape))

    (wh0, wh1, s0, s1, t0p, t0b, t1p, t1b, t0m, t1m) = pl.pallas_call(
        _prelude_kernel,
        grid=(_NI,),
        in_specs=[
            pl.BlockSpec((_BI, _IN_F), lambda i: (i, 0)),
            const((_IN_F, _IN_F)), const((1, _IN_F)),
            const((_IN_F, _HID)), const((_IN_F, _HID)),
            const((_HID, 1)), const((_HID, 1)),
            const((_HID, 1)), const((_HID, 1)),
        ],
        out_specs=[
            pl.BlockSpec((_BI, _EXT), lambda i: (i, 0)),
            pl.BlockSpec((_BI, _EXT), lambda i: (i, 0)),
            pl.BlockSpec((_BI, 1), lambda i: (i, 0)),
            pl.BlockSpec((_BI, 1), lambda i: (i, 0)),
            pl.BlockSpec((2, _BI), lambda i: (0, i)),
            pl.BlockSpec((2, _BI), lambda i: (0, i)),
            pl.BlockSpec((2, _BI), lambda i: (0, i)),
            pl.BlockSpec((2, _BI), lambda i: (0, i)),
            const((1, 1)), const((1, 1)),
        ],
        out_shape=[
            jax.ShapeDtypeStruct((_NC, _EXT), _bf16),
            jax.ShapeDtypeStruct((_NC, _EXT), _bf16),
            jax.ShapeDtypeStruct((_NC, 1), _f32),
            jax.ShapeDtypeStruct((_NC, 1), _f32),
            jax.ShapeDtypeStruct((2, _NC), _bf16),
            jax.ShapeDtypeStruct((2, _NC), _bf16),
            jax.ShapeDtypeStruct((2, _NC), _bf16),
            jax.ShapeDtypeStruct((2, _NC), _bf16),
            jax.ShapeDtypeStruct((1, 1), _f32),
            jax.ShapeDtypeStruct((1, 1), _f32),
        ],
        scratch_shapes=[pltpu.VMEM((1, 1), _f32), pltpu.VMEM((1, 1), _f32)],
    )(x, W_lin, blin, w0, w1, a01, a02, a11, a12)

    whe, se, tep, teb, tem, pk = pl.pallas_call(
        _pass1_kernel,
        grid=(_NI, _NJ),
        in_specs=[
            pl.BlockSpec((_BI, _BJ), lambda i, j: (i, j)),
            const((_BG, _BI)),
            pl.BlockSpec((_BJ, _EXT), lambda i, j: (j, 0)),
            pl.BlockSpec((_BJ, _EXT), lambda i, j: (j, 0)),
            pl.BlockSpec((_BI, 1), lambda i, j: (i, 0)),
            pl.BlockSpec((_BI, 1), lambda i, j: (i, 0)),
            pl.BlockSpec((2, _BJ), lambda i, j: (0, j)),
            pl.BlockSpec((2, _BJ), lambda i, j: (0, j)),
            pl.BlockSpec((2, _BJ), lambda i, j: (0, j)),
            pl.BlockSpec((2, _BJ), lambda i, j: (0, j)),
            const((1, 1)), const((1, 1)),
            const((_IN_F, _OUT)),
            const((_OUT, 1)), const((_OUT, 1)),
        ],
        out_specs=[
            pl.BlockSpec((_BI, _EXT), lambda i, j: (i, 0)),
            pl.BlockSpec((_BI, 1), lambda i, j: (i, 0)),
            pl.BlockSpec((2, _BI), lambda i, j: (0, i)),
            pl.BlockSpec((2, _BI), lambda i, j: (0, i)),
            const((1, 1)),
            pl.BlockSpec((_BG, _BJ), lambda i, j: (i, j)),
        ],
        out_shape=[
            jax.ShapeDtypeStruct((_NC, _EXT), _bf16),
            jax.ShapeDtypeStruct((_NC, 1), _f32),
            jax.ShapeDtypeStruct((2, _NC), _bf16),
            jax.ShapeDtypeStruct((2, _NC), _bf16),
            jax.ShapeDtypeStruct((1, 1), _f32),
            jax.ShapeDtypeStruct((_NP, _NC), jnp.uint8),
        ],
        scratch_shapes=[
            pltpu.VMEM((_BI, _EXT), _f32), pltpu.VMEM((_BI, _EXT), _f32),
            pltpu.VMEM((1, 1), _f32),
        ],
    )(adj, apack, wh0, wh1, s0, s1, t0p, t0b, t1p, t1b, t0m, t1m,
      W_end, ae1, ae2)

    out = pl.pallas_call(
        _pass2_kernel,
        grid=(_NI, _NJ),
        in_specs=[
            pl.BlockSpec((_BG, _BJ), lambda i, j: (i, j)),
            pl.BlockSpec((_BJ, _EXT), lambda i, j: (j, 0)),
            pl.BlockSpec((_BI, 1), lambda i, j: (i, 0)),
            pl.BlockSpec((2, _BJ), lambda i, j: (0, j)),
            pl.BlockSpec((2, _BJ), lambda i, j: (0, j)),
            const((1, 1)),
        ],
        out_specs=pl.BlockSpec((_BI, _OUT), lambda i, j: (i, 0)),
        out_shape=jax.ShapeDtypeStruct((_N, _OUT), _f32),
        scratch_shapes=[pltpu.VMEM((_BI, _EXT), _f32)],
    )(pk, whe, se, tep, teb, tem)

    return out


# speedup vs baseline: 1.1632x; 1.1632x over previous
import jax
import jax.numpy as jnp
from jax.experimental import pallas as pl
from jax.experimental.pallas import tpu as pltpu

_N = 10000
_IN_F = 128
_HID = 64
_OUT = 64
_ALPHA = 0.2
_NEGBIG = -1e30
_LOG2E = 1.4426950408889634

_BI = 2048
_BJ = 2048
_NI = (_N + _BI - 1) // _BI
_NJ = (_N + _BJ - 1) // _BJ
_BG = _BI // 8
_NP = _NI * _BG
_NC = _NJ * _BJ
_EXT = 128

_f32 = jnp.float32
_bf16 = jnp.bfloat16


def _row_t(a2, wh):
    return jax.lax.dot_general(a2, wh, (((0,), (1,)), ((), ())),
                               preferred_element_type=_f32)


def _extend(wh, rowv):
    b = wh.shape[0]
    ext = jnp.concatenate(
        [wh, jnp.ones((b, 1), _f32), jnp.zeros((b, _EXT - _HID - 1), _f32)],
        axis=1)
    return jnp.where(rowv, ext, 0.0).astype(_bf16)


def _prep_st(wh, a1, a2, colv):
    b = wh.shape[0]
    s = jnp.dot(wh, a1, preferred_element_type=_f32) * _LOG2E
    t = _row_t(a2, wh) * _LOG2E
    ones = jnp.ones((1, b), _f32)
    tp = jnp.concatenate([ones, jnp.where(colv, t, _NEGBIG)], axis=0).astype(_bf16)
    tb = jnp.concatenate([ones, jnp.where(colv, t * _ALPHA, _NEGBIG)], axis=0).astype(_bf16)
    tm = jnp.max(jnp.where(colv, t, _NEGBIG), keepdims=True)
    return s, tp, tb, tm


def _lrelu(z):
    return jnp.maximum(z, _ALPHA * z)


def _prelude_kernel(x_ref, wlin_ref, blin_ref, w0_ref, w1_ref,
                    a01_ref, a02_ref, a11_ref, a12_ref,
                    wh0_o, wh1_o, s0_o, s1_o, t0p_o, t0b_o, t1p_o, t1b_o,
                    t0m_o, t1m_o, t0m_s, t1m_s):
    i = pl.program_id(0)
    h = jnp.dot(x_ref[...], wlin_ref[...],
                preferred_element_type=_f32) + blin_ref[...]
    wh0 = jnp.dot(h, w0_ref[...], preferred_element_type=_f32)
    wh1 = jnp.dot(h, w1_ref[...], preferred_element_type=_f32)
    rowv = (i * _BI + jax.lax.broadcasted_iota(jnp.int32, (_BI, 1), 0)) < _N
    colv = (i * _BI + jax.lax.broadcasted_iota(jnp.int32, (1, _BI), 1)) < _N
    wh0_o[...] = _extend(wh0, rowv)
    wh1_o[...] = _extend(wh1, rowv)
    s0, t0p, t0b, bm0 = _prep_st(wh0, a01_ref[...], a02_ref[...], colv)
    s1, t1p, t1b, bm1 = _prep_st(wh1, a11_ref[...], a12_ref[...], colv)
    s0_o[...] = s0
    s1_o[...] = s1
    t0p_o[...] = t0p
    t0b_o[...] = t0b
    t1p_o[...] = t1p
    t1b_o[...] = t1b
    prev0 = jnp.where(i == 0, jnp.full((1, 1), _NEGBIG), t0m_s[...])
    prev1 = jnp.where(i == 0, jnp.full((1, 1), _NEGBIG), t1m_s[...])
    t0m_s[...] = jnp.maximum(prev0, bm0)
    t1m_s[...] = jnp.maximum(prev1, bm1)

    @pl.when(i == _NI - 1)
    def _():
        t0m_o[...] = t0m_s[...]
        t1m_o[...] = t1m_s[...]


def _head_step(okb, s_ref, tp_ref, tb_ref, tm_ref, wh_ref, acc):
    s = s_ref[...]
    m = _lrelu(s + tm_ref[...])
    b = s.shape[0]
    ones = jnp.ones((b, 1), _f32)
    ea = (s - m).astype(_bf16) + tp_ref[1:2, :]
    eb = (s * _ALPHA - m).astype(_bf16) + tb_ref[1:2, :]
    x = jnp.exp2(jnp.maximum(ea, eb))
    p = jnp.where(okb, x, _bf16(0.0))
    acc[...] += jnp.dot(p, wh_ref[...], preferred_element_type=_f32)


def _pass1_kernel(adj_ref, apack_ref, wh0_ref, wh1_ref, s0_ref, s1_ref,
                  t0p_ref, t0b_ref, t1p_ref, t1b_ref, t0m_ref, t1m_ref,
                  wend_ref, ae1_ref, ae2_ref,
                  whe_o, se_o, tep_o, teb_o, tem_o, pk_o,
                  acc0, acc1, tem_s):
    i = pl.program_id(0)
    j = pl.program_id(1)

    @pl.when(j == 0)
    def _():
        acc0[...] = jnp.zeros_like(acc0)
        acc1[...] = jnp.zeros_like(acc1)

    adjb = adj_ref[...] > 0

    pk = adj_ref[0:_BG, :]
    for r in range(1, 8):
        pk += adj_ref[r * _BG:(r + 1) * _BG, :] << r
    pk_o[...] = pk.astype(jnp.uint8)

    okb = adjb

    _head_step(okb, s0_ref, t0p_ref, t0b_ref, t0m_ref, wh0_ref, acc0)
    _head_step(okb, s1_ref, t1p_ref, t1b_ref, t1m_ref, wh1_ref, acc1)

    @pl.when(j == _NJ - 1)
    def _():
        h0 = acc0[:, :_HID] / jnp.maximum(acc0[:, _HID:_HID + 1], 1e-30)
        h1 = acc1[:, :_HID] / jnp.maximum(acc1[:, _HID:_HID + 1], 1e-30)
        x0 = jnp.where(h0 > 0, h0, jnp.exp(h0) - 1.0)
        x1 = jnp.where(h1 > 0, h1, jnp.exp(h1) - 1.0)
        whe = (jnp.dot(x0, wend_ref[:_HID, :], preferred_element_type=_f32)
               + jnp.dot(x1, wend_ref[_HID:, :], preferred_element_type=_f32))
        rowv = (i * _BI + jax.lax.broadcasted_iota(jnp.int32, (_BI, 1), 0)) < _N
        colv = (i * _BI + jax.lax.broadcasted_iota(jnp.int32, (1, _BI), 1)) < _N
        whe_o[...] = _extend(whe, rowv)
        se, tep, teb, bm = _prep_st(whe, ae1_ref[...], ae2_ref[...], colv)
        se_o[...] = se
        tep_o[...] = tep
        teb_o[...] = teb
        prev = jnp.where(i == 0, jnp.full((1, 1), _NEGBIG), tem_s[...])
        tem_s[...] = jnp.maximum(prev, bm)

        @pl.when(i == _NI - 1)
        def _():
            tem_o[...] = tem_s[...]


def _pass2_kernel(pk_ref, whe_ref, se_ref, tep_ref, teb_ref, tem_ref,
                  out_o, acc):
    j = pl.program_id(1)

    @pl.when(j == 0)
    def _():
        acc[...] = jnp.zeros_like(acc)

    pk = pk_ref[...].astype(jnp.int32)
    s = se_ref[...]
    m = _lrelu(s + tem_ref[...])
    sa = (s - m).astype(_bf16)
    sb = (s * _ALPHA - m).astype(_bf16)
    tp = tep_ref[1:2, :]
    tb = teb_ref[1:2, :]
    for r in range(8):
        lo, hi = r * _BG, (r + 1) * _BG
        ea = sa[lo:hi] + tp
        eb = sb[lo:hi] + tb
        x = jnp.exp2(jnp.maximum(ea, eb))
        p = jnp.where(((pk >> r) & 1) > 0, x, _bf16(0.0))
        acc[lo:hi, :] += jnp.dot(p, whe_ref[...],
                                 preferred_element_type=_f32)

    @pl.when(j == _NJ - 1)
    def _():
        o = acc[:, :_OUT] / jnp.maximum(acc[:, _OUT:_OUT + 1], 1e-30)
        z = o - jnp.max(o, axis=1, keepdims=True)
        pz = jnp.exp(z)
        out_o[...] = pz / jnp.sum(pz, axis=1, keepdims=True)


def kernel(x, adj, W_lin, b_lin, W_heads, a_heads, W_end, a_end):
    w0, w1 = W_heads[0], W_heads[1]
    a01, a02 = a_heads[0, :_HID], a_heads[0, _HID:]
    a11, a12 = a_heads[1, :_HID], a_heads[1, _HID:]
    ae1, ae2 = a_end[:_OUT], a_end[_OUT:]
    blin = b_lin.reshape(1, _IN_F)
    cols = jnp.arange(_BJ)
    apack = (((cols % _BG)[None, :] == jnp.arange(_BG)[:, None])
             * (2.0 ** (cols // _BG))[None, :]).astype(_bf16)

    const = lambda shape: pl.BlockSpec(shape, lambda *_: tuple(0 for _ in shape))

    (wh0, wh1, s0, s1, t0p, t0b, t1p, t1b, t0m, t1m) = pl.pallas_call(
        _prelude_kernel,
        grid=(_NI,),
        in_specs=[
            pl.BlockSpec((_BI, _IN_F), lambda i: (i, 0)),
            const((_IN_F, _IN_F)), const((1, _IN_F)),
            const((_IN_F, _HID)), const((_IN_F, _HID)),
            const((_HID, 1)), const((_HID, 1)),
            const((_HID, 1)), const((_HID, 1)),
        ],
        out_specs=[
            pl.BlockSpec((_BI, _EXT), lambda i: (i, 0)),
            pl.BlockSpec((_BI, _EXT), lambda i: (i, 0)),
            pl.BlockSpec((_BI, 1), lambda i: (i, 0)),
            pl.BlockSpec((_BI, 1), lambda i: (i, 0)),
            pl.BlockSpec((2, _BI), lambda i: (0, i)),
            pl.BlockSpec((2, _BI), lambda i: (0, i)),
            pl.BlockSpec((2, _BI), lambda i: (0, i)),
            pl.BlockSpec((2, _BI), lambda i: (0, i)),
            const((1, 1)), const((1, 1)),
        ],
        out_shape=[
            jax.ShapeDtypeStruct((_NC, _EXT), _bf16),
            jax.ShapeDtypeStruct((_NC, _EXT), _bf16),
            jax.ShapeDtypeStruct((_NC, 1), _f32),
            jax.ShapeDtypeStruct((_NC, 1), _f32),
            jax.ShapeDtypeStruct((2, _NC), _bf16),
            jax.ShapeDtypeStruct((2, _NC), _bf16),
            jax.ShapeDtypeStruct((2, _NC), _bf16),
            jax.ShapeDtypeStruct((2, _NC), _bf16),
            jax.ShapeDtypeStruct((1, 1), _f32),
            jax.ShapeDtypeStruct((1, 1), _f32),
        ],
        scratch_shapes=[pltpu.VMEM((1, 1), _f32), pltpu.VMEM((1, 1), _f32)],
    )(x, W_lin, blin, w0, w1, a01, a02, a11, a12)

    whe, se, tep, teb, tem, pk = pl.pallas_call(
        _pass1_kernel,
        grid=(_NI, _NJ),
        in_specs=[
            pl.BlockSpec((_BI, _BJ), lambda i, j: (i, j)),
            const((_BG, _BI)),
            pl.BlockSpec((_BJ, _EXT), lambda i, j: (j, 0)),
            pl.BlockSpec((_BJ, _EXT), lambda i, j: (j, 0)),
            pl.BlockSpec((_BI, 1), lambda i, j: (i, 0)),
            pl.BlockSpec((_BI, 1), lambda i, j: (i, 0)),
            pl.BlockSpec((2, _BJ), lambda i, j: (0, j)),
            pl.BlockSpec((2, _BJ), lambda i, j: (0, j)),
            pl.BlockSpec((2, _BJ), lambda i, j: (0, j)),
            pl.BlockSpec((2, _BJ), lambda i, j: (0, j)),
            const((1, 1)), const((1, 1)),
            const((_IN_F, _OUT)),
            const((_OUT, 1)), const((_OUT, 1)),
        ],
        out_specs=[
            pl.BlockSpec((_BI, _EXT), lambda i, j: (i, 0)),
            pl.BlockSpec((_BI, 1), lambda i, j: (i, 0)),
            pl.BlockSpec((2, _BI), lambda i, j: (0, i)),
            pl.BlockSpec((2, _BI), lambda i, j: (0, i)),
            const((1, 1)),
            pl.BlockSpec((_BG, _BJ), lambda i, j: (i, j)),
        ],
        out_shape=[
            jax.ShapeDtypeStruct((_NC, _EXT), _bf16),
            jax.ShapeDtypeStruct((_NC, 1), _f32),
            jax.ShapeDtypeStruct((2, _NC), _bf16),
            jax.ShapeDtypeStruct((2, _NC), _bf16),
            jax.ShapeDtypeStruct((1, 1), _f32),
            jax.ShapeDtypeStruct((_NP, _NC), jnp.uint8),
        ],
        scratch_shapes=[
            pltpu.VMEM((_BI, _EXT), _f32), pltpu.VMEM((_BI, _EXT), _f32),
            pltpu.VMEM((1, 1), _f32),
        ],
    )(adj, apack, wh0, wh1, s0, s1, t0p, t0b, t1p, t1b, t0m, t1m,
      W_end, ae1, ae2)

    out = pl.pallas_call(
        _pass2_kernel,
        grid=(_NI, _NJ),
        in_specs=[
            pl.BlockSpec((_BG, _BJ), lambda i, j: (i, j)),
            pl.BlockSpec((_BJ, _EXT), lambda i, j: (j, 0)),
            pl.BlockSpec((_BI, 1), lambda i, j: (i, 0)),
            pl.BlockSpec((2, _BJ), lambda i, j: (0, j)),
            pl.BlockSpec((2, _BJ), lambda i, j: (0, j)),
            const((1, 1)),
        ],
        out_specs=pl.BlockSpec((_BI, _OUT), lambda i, j: (i, 0)),
        out_shape=jax.ShapeDtypeStruct((_N, _OUT), _f32),
        scratch_shapes=[pltpu.VMEM((_BI, _EXT), _f32)],
    )(pk, whe, se, tep, teb, tem)

    return out
